# Initial kernel scaffold; baseline (speedup 1.0000x reference)
import jax
import jax.numpy as jnp
from jax.experimental import pallas as pl


def kernel(depth, context, input_feature_map, bin_num, min_depth, max_depth, masks, instances, boxes, labels, W_ss, b_ss, W_can, b_can):
    raise NotImplementedError("write your pallas kernel here")


# speedup vs baseline: 431.2086x; 431.2086x over previous
import functools

import jax
import jax.numpy as jnp
from jax import lax
from jax.experimental import pallas as pl

_PREC = lax.Precision.HIGHEST
_S = 7


def _contract_body(w_ref, f_ref, g_ref):
    g_ref[0] = jnp.dot(w_ref[...], f_ref[0], precision=_PREC,
                       preferred_element_type=jnp.float32)


def _assemble_body(g_ref, par_ref, const_ref, t_ref, d_ref, can_ref,
                   sc_ref, sh_ref, *, h, w, Hf, Wf):
    f32 = jnp.float32
    b0 = par_ref[0, 0, 0]
    b1 = par_ref[0, 0, 1]
    b2 = par_ref[0, 0, 2]
    b3 = par_ref[0, 0, 3]
    validf = par_ref[0, 0, 4]
    b_can0 = const_ref[0, 0, 0]
    b_ss0 = const_ref[0, 0, 1]
    b_ss1 = const_ref[0, 0, 2]

    eps = jnp.float32(1e-3)
    x1 = jnp.minimum(b0, b2)
    x2 = jnp.maximum(b0, b2) + eps
    y1 = jnp.minimum(b1, b3)
    y2 = jnp.maximum(b1, b3) + eps

    tmat = t_ref[0]
    ys = y1 + (y2 - y1) * tmat
    xs = x1 + (x2 - x1) * tmat
    iyf = jnp.clip(jnp.round(ys * (Hf - 1)), 0, Hf - 1)
    ixf = jnp.clip(jnp.round(xs * (Wf - 1)), 0, Wf - 1)
    col = lax.broadcasted_iota(jnp.int32, (8, Wf), 1).astype(f32)
    row_ok = lax.broadcasted_iota(jnp.int32, (8, Wf), 0) < _S
    Ry = ((col == iyf) & row_ok).astype(f32)
    Rx = ((col == ixf) & row_ok).astype(f32)

    def samp(m):
        a = jnp.dot(Ry, m, precision=_PREC, preferred_element_type=f32)
        return lax.dot_general(a, Rx, (((1,), (1,)), ((), ())),
                               precision=_PREC, preferred_element_type=f32)

    g_can = g_ref[0, 0]
    g_s0 = g_ref[0, 1]
    g_s1 = g_ref[0, 2]
    T_can = samp(g_can)
    T_s0 = samp(g_s0)
    T_s1 = samp(g_s1)

    inv49 = jnp.float32(1.0 / 49.0)
    scale = (jnp.sum(T_s0) * inv49 + b_ss0) * validf
    shift = (jnp.sum(T_s1) * inv49 + b_ss1) * validf
    D8 = T_can + b_can0

    yv = lax.broadcasted_iota(jnp.int32, (h, 8), 0).astype(f32) / (h - 1)
    uy = (yv - y1) / (y2 - y1)
    gyf = jnp.clip(jnp.round(uy * (_S - 1)), 0, _S - 1)
    kA = lax.broadcasted_iota(jnp.int32, (h, 8), 1).astype(f32)
    Ay = ((kA == gyf) & (uy >= 0) & (uy <= 1)).astype(f32)

    xv = lax.broadcasted_iota(jnp.int32, (8, w), 1).astype(f32) / (w - 1)
    ux = (xv - x1) / (x2 - x1)
    gxf = jnp.clip(jnp.round(ux * (_S - 1)), 0, _S - 1)
    kB = lax.broadcasted_iota(jnp.int32, (8, w), 0).astype(f32)
    Bx = ((kB == gxf) & (ux >= 0) & (ux <= 1)).astype(f32)

    E = jnp.dot(D8, Bx, precision=_PREC, preferred_element_type=f32)
    can = jnp.dot(Ay, E, precision=_PREC, preferred_element_type=f32)
    can = can * validf

    d = jnp.clip(jax.nn.relu(can * scale + shift), 0.001, None)

    can_ref[0, 0] = can
    d_ref[0, 0] = d
    sc_ref[...] = jnp.full((1, 8, 128), scale, f32)
    sh_ref[...] = jnp.full((1, 8, 128), shift, f32)


def kernel(depth, context, input_feature_map, bin_num, min_depth, max_depth,
           masks, instances, boxes, labels, W_ss, b_ss, W_can, b_can):
    f32 = jnp.float32
    B, I, h, w = instances.shape
    _, C, Hf, Wf = input_feature_map.shape
    N = B * I

    Wcat = jnp.concatenate(
        [W_can[:, 0:1], W_ss[:, 0:1], W_ss[:, 1:2],
         jnp.zeros((C, 5), f32)], axis=1).T
    fmap2 = input_feature_map.reshape(B, C, Hf * Wf)
    g = pl.pallas_call(
        _contract_body,
        grid=(B,),
        in_specs=[
            pl.BlockSpec((8, C), lambda b: (0, 0)),
            pl.BlockSpec((1, C, Hf * Wf), lambda b: (b, 0, 0)),
        ],
        out_specs=pl.BlockSpec((1, 8, Hf * Wf), lambda b: (b, 0, 0)),
        out_shape=jax.ShapeDtypeStruct((B, 8, Hf * Wf), f32),
    )(Wcat, fmap2)
    g = g.reshape(B, 8, Hf, Wf)

    valid = (labels.reshape(N, 1) != 0).astype(f32)
    params = jnp.concatenate(
        [boxes.reshape(N, 4), valid, jnp.zeros((N, 3), f32)],
        axis=1).reshape(N, 1, 8)
    consts = jnp.concatenate(
        [b_can.reshape(-1), b_ss.reshape(-1),
         jnp.zeros((5,), f32)]).reshape(1, 1, 8)
    t = jnp.linspace(0.0, 1.0, _S).astype(f32)
    tmat = jnp.broadcast_to(
        jnp.concatenate([t, jnp.zeros((1,), f32)])[:, None],
        (8, Wf)).reshape(1, 8, Wf)

    body = functools.partial(_assemble_body, h=h, w=w, Hf=Hf, Wf=Wf)
    d, can, scale, shift = pl.pallas_call(
        body,
        grid=(N,),
        in_specs=[
            pl.BlockSpec((1, 8, Hf, Wf), lambda v: (v // I, 0, 0, 0)),
            pl.BlockSpec((1, 1, 8), lambda v: (v, 0, 0)),
            pl.BlockSpec((1, 1, 8), lambda v: (0, 0, 0)),
            pl.BlockSpec((1, 8, Wf), lambda v: (0, 0, 0)),
        ],
        out_specs=[
            pl.BlockSpec((1, 1, h, w), lambda v: (v // I, v % I, 0, 0)),
            pl.BlockSpec((1, 1, h, w), lambda v: (v // I, v % I, 0, 0)),
            pl.BlockSpec((1, 8, 128), lambda v: (v, 0, 0)),
            pl.BlockSpec((1, 8, 128), lambda v: (v, 0, 0)),
        ],
        out_shape=[
            jax.ShapeDtypeStruct((B, I, h, w), f32),
            jax.ShapeDtypeStruct((B, I, h, w), f32),
            jax.ShapeDtypeStruct((N, 8, 128), f32),
            jax.ShapeDtypeStruct((N, 8, 128), f32),
        ],
    )(g, params, consts, tmat)

    return (d, can, scale[:, 0, 0].reshape(B, I), shift[:, 0, 0].reshape(B, I))
